# Initial kernel scaffold; baseline (speedup 1.0000x reference)
import jax
import jax.numpy as jnp
from jax.experimental import pallas as pl


def kernel(x, edge_index, W1, b1, W2, b2):
    raise NotImplementedError("write your pallas kernel here")


# speedup vs baseline: 14.2248x; 14.2248x over previous
import functools

import jax
import jax.numpy as jnp
from jax import lax
from jax.experimental import pallas as pl
from jax.experimental.pallas import tpu as pltpu
from jax.experimental.pallas import tpu_sc as plsc

N = 10000
NPAD = 10240
E = 320000
D_IN = 128
D_HID = 64
D_OUT = 64

NC = 2
NS = 16
NW = NC * NS
CH = 128
NCHUNK = 79
EPT = NCHUNK * CH
EPAD = NW * EPT
RPT = NPAD // NS

RB = 1024


def _mesh():
    return plsc.VectorSubcoreMesh(core_axis_name="c", subcore_axis_name="s")


@functools.partial(
    pl.kernel,
    mesh=_mesh(),
    compiler_params=pltpu.CompilerParams(use_tc_tiling_on_sc=False),
    out_type=jax.ShapeDtypeStruct((NC * NPAD,), jnp.float32),
    scratch_types=[
        pltpu.VMEM((CH,), jnp.int32),
        pltpu.VMEM((CH,), jnp.float32),
        pltpu.VMEM_SHARED((NPAD,), jnp.float32),
    ],
)
def _sc_degree(dst_hbm, ones_hbm, zrow_hbm, out_hbm, dst_v, ones_v, acc_sh):
    c = lax.axis_index("c")
    s = lax.axis_index("s")
    pltpu.sync_copy(zrow_hbm, acc_sh.at[pl.ds(s * RPT, RPT)])
    pltpu.sync_copy(ones_hbm, ones_v)
    plsc.subcore_barrier()
    base = (s * NC + c) * EPT

    def body(i, carry):
        off = base + i * CH
        pltpu.sync_copy(dst_hbm.at[pl.ds(off, CH)], dst_v)
        pltpu.sync_copy(ones_v, acc_sh.at[dst_v], add=True)
        return carry

    lax.fori_loop(0, NCHUNK, body, 0)
    plsc.subcore_barrier()
    pltpu.sync_copy(acc_sh.at[pl.ds(s * RPT, RPT)],
                    out_hbm.at[pl.ds(c * NPAD + s * RPT, RPT)])


@functools.partial(
    pl.kernel,
    mesh=_mesh(),
    compiler_params=pltpu.CompilerParams(use_tc_tiling_on_sc=False),
    out_type=jax.ShapeDtypeStruct((NC * NPAD, D_HID), jnp.float32),
    scratch_types=[
        pltpu.VMEM((CH,), jnp.int32),
        pltpu.VMEM((CH,), jnp.int32),
        pltpu.VMEM((CH, D_HID), jnp.float32),
        pltpu.VMEM_SHARED((NPAD, D_HID), jnp.float32),
        pltpu.SemaphoreType.DMA,
    ],
)
def _sc_aggregate(hp_hbm, src_hbm, dst_hbm, zblk_hbm, out_hbm,
                  src_v, dst_v, rows_v, acc_sh, sem):
    c = lax.axis_index("c")
    s = lax.axis_index("s")
    pltpu.sync_copy(zblk_hbm, acc_sh.at[pl.ds(s * RPT, RPT)])
    plsc.subcore_barrier()
    base = (s * NC + c) * EPT

    def body(i, carry):
        off = base + i * CH
        pltpu.sync_copy(src_hbm.at[pl.ds(off, CH)], src_v)
        pltpu.sync_copy(dst_hbm.at[pl.ds(off, CH)], dst_v)
        pltpu.async_copy(hp_hbm.at[src_v], rows_v, sem).wait()
        pltpu.sync_copy(rows_v, acc_sh.at[dst_v], add=True)
        return carry

    lax.fori_loop(0, NCHUNK, body, 0)
    plsc.subcore_barrier()
    pltpu.sync_copy(acc_sh.at[pl.ds(s * RPT, RPT)],
                    out_hbm.at[pl.ds(c * NPAD + s * RPT, RPT)])


def _b_body(x_ref, w_ref, dg_ref, h_ref, dinv_ref):
    deg = dg_ref[:, 0:1] + dg_ref[:, 1:2] + 1.0
    dinv = lax.rsqrt(jnp.maximum(deg, 1.0))
    dinv_ref[...] = dinv
    h_ref[...] = jnp.dot(x_ref[...], w_ref[...],
                         preferred_element_type=jnp.float32) * dinv


def _tc_layer1(x_pad, W1, degT):
    return pl.pallas_call(
        _b_body,
        grid=(NPAD // RB,),
        in_specs=[
            pl.BlockSpec((RB, D_IN), lambda i: (i, 0)),
            pl.BlockSpec((D_IN, D_HID), lambda i: (0, 0)),
            pl.BlockSpec((RB, NC), lambda i: (i, 0)),
        ],
        out_specs=[
            pl.BlockSpec((RB, D_HID), lambda i: (i, 0)),
            pl.BlockSpec((RB, 1), lambda i: (i, 0)),
        ],
        out_shape=[
            jax.ShapeDtypeStruct((NPAD, D_HID), jnp.float32),
            jax.ShapeDtypeStruct((NPAD, 1), jnp.float32),
        ],
    )(x_pad, W1, degT)


def _d_body(a0, a1, hp, dinv, b1, w2, out):
    agg = (a0[...] + a1[...] + hp[...]) * dinv[...] + b1[...]
    h2 = jnp.maximum(agg, 0.0)
    out[...] = jnp.dot(h2, w2[...], preferred_element_type=jnp.float32) * dinv[...]


def _tc_layer2(a0, a1, h1p, dinv, b1, W2):
    return pl.pallas_call(
        _d_body,
        grid=(NPAD // RB,),
        in_specs=[
            pl.BlockSpec((RB, D_HID), lambda i: (i, 0)),
            pl.BlockSpec((RB, D_HID), lambda i: (i, 0)),
            pl.BlockSpec((RB, D_HID), lambda i: (i, 0)),
            pl.BlockSpec((RB, 1), lambda i: (i, 0)),
            pl.BlockSpec((1, D_HID), lambda i: (0, 0)),
            pl.BlockSpec((D_HID, D_OUT), lambda i: (0, 0)),
        ],
        out_specs=pl.BlockSpec((RB, D_OUT), lambda i: (i, 0)),
        out_shape=jax.ShapeDtypeStruct((NPAD, D_OUT), jnp.float32),
    )(a0, a1, h1p, dinv, b1, W2)


def _f_body(a0, a1, hp, dinv, b2, out):
    out[...] = (a0[...] + a1[...] + hp[...]) * dinv[...] + b2[...]


def _tc_final(a0, a1, h2p, dinv, b2):
    return pl.pallas_call(
        _f_body,
        grid=(NPAD // RB,),
        in_specs=[
            pl.BlockSpec((RB, D_OUT), lambda i: (i, 0)),
            pl.BlockSpec((RB, D_OUT), lambda i: (i, 0)),
            pl.BlockSpec((RB, D_OUT), lambda i: (i, 0)),
            pl.BlockSpec((RB, 1), lambda i: (i, 0)),
            pl.BlockSpec((1, D_OUT), lambda i: (0, 0)),
        ],
        out_specs=pl.BlockSpec((RB, D_OUT), lambda i: (i, 0)),
        out_shape=jax.ShapeDtypeStruct((NPAD, D_OUT), jnp.float32),
    )(a0, a1, h2p, dinv, b2)


def kernel(x, edge_index, W1, b1, W2, b2):
    ei = edge_index.astype(jnp.int32)
    src = jnp.concatenate([ei[0], jnp.zeros((EPAD - E,), jnp.int32)])
    dst = jnp.concatenate([ei[1], jnp.full((EPAD - E,), N, jnp.int32)])
    x_pad = jnp.concatenate([x, jnp.zeros((NPAD - N, D_IN), x.dtype)])
    ones_ch = jnp.ones((CH,), jnp.float32)
    zrow = jnp.zeros((RPT,), jnp.float32)
    zblk = jnp.zeros((RPT, D_HID), jnp.float32)

    deg2 = _sc_degree(dst, ones_ch, zrow).reshape(NC, NPAD)
    degT = deg2.T
    h1p, dinv = _tc_layer1(x_pad, W1, degT)
    agg1 = _sc_aggregate(h1p, src, dst, zblk).reshape(NC, NPAD, D_HID)
    h2p = _tc_layer2(agg1[0], agg1[1], h1p, dinv, b1.reshape(1, D_HID), W2)
    agg2 = _sc_aggregate(h2p, src, dst, zblk).reshape(NC, NPAD, D_HID)
    outp = _tc_final(agg2[0], agg2[1], h2p, dinv, b2.reshape(1, D_OUT))
    return outp[:N]
